# baseline (speedup 1.0000x reference)
import functools
import math

import jax
import jax.numpy as jnp
from jax import lax
from jax.experimental import pallas as pl
from jax.experimental.pallas import tpu as pltpu
from jax.experimental.pallas import tpu_sc as plsc

_BATCH = 16384
_NW = 32
_BPW = _BATCH // _NW
_HALF = _BPW // 2
_LANES = 16
_C = math.log(10.0) / 400.0


def _bt_body(idxc_hbm, elos_hbm, out_hbm, ic_v, e_v, o_v, ssem, gsem0, gsem1, osem):
    wid = lax.axis_index("s") * 2 + lax.axis_index("c")
    base = wid * _BPW

    pltpu.async_copy(idxc_hbm.at[pl.ds(2 * base, 2 * _BPW)], ic_v, ssem).wait()

    pltpu.async_copy(elos_hbm.at[ic_v], e_v, gsem0).wait()

    def _compute(k, carry):
        a = e_v[pl.ds(k * _LANES, _LANES)]
        b = e_v[pl.ds(_BPW + k * _LANES, _LANES)]
        e = jnp.exp((b - a) * _C)
        o_v[pl.ds(k * _LANES, _LANES)] = 1.0 / (1.0 + e)
        return carry

    lax.fori_loop(0, _BPW // _LANES, _compute, 0)
    pltpu.async_copy(o_v, out_hbm.at[pl.ds(base, _BPW)], osem).wait()


@jax.jit
def kernel(idx_a, idx_b, elos):
    mesh = plsc.VectorSubcoreMesh(core_axis_name="c", subcore_axis_name="s")
    run = functools.partial(
        pl.kernel,
        mesh=mesh,
        out_type=jax.ShapeDtypeStruct((_BATCH,), jnp.float32),
        scratch_types=[
            pltpu.VMEM((2 * _BPW,), jnp.int32),
            pltpu.VMEM((2 * _BPW,), jnp.float32),
            pltpu.VMEM((_BPW,), jnp.float32),
            pltpu.SemaphoreType.DMA,
            pltpu.SemaphoreType.DMA,
            pltpu.SemaphoreType.DMA,
            pltpu.SemaphoreType.DMA,
        ],
    )(_bt_body)
    a = idx_a.astype(jnp.int32).reshape(_NW, _BPW)
    b = idx_b.astype(jnp.int32).reshape(_NW, _BPW)
    idxc = jnp.stack([a, b], axis=1).reshape(-1)
    return run(idxc, elos)


# speedup vs baseline: 1.0167x; 1.0167x over previous
import functools
import math

import jax
import jax.numpy as jnp
from jax import lax
from jax.experimental import pallas as pl
from jax.experimental.pallas import tpu as pltpu
from jax.experimental.pallas import tpu_sc as plsc

_BATCH = 16384
_NW = 32
_BPW = _BATCH // _NW
_HALF = _BPW // 2
_LANES = 16
_C = math.log(10.0) / 400.0


def _bt_body(idxc_hbm, elos_hbm, out_hbm, ic_v, e_v, ssem, gsem0, osem):
    wid = lax.axis_index("s") * 2 + lax.axis_index("c")
    base = wid * _BPW

    pltpu.async_copy(idxc_hbm.at[pl.ds(2 * base, 2 * _BPW)], ic_v, ssem).wait()

    gsems = (gsem0, ssem)
    gathers = [
        pltpu.async_copy(
            elos_hbm.at[ic_v.at[pl.ds(h * _BPW, _BPW)]],
            e_v.at[pl.ds(h * _BPW, _BPW)], gsems[h])
        for h in range(2)
    ]

    stores = []
    for h in range(2):
        gathers[h].wait()
        for k in range(_HALF // _LANES):
            a = e_v[pl.ds(h * _BPW + k * _LANES, _LANES)]
            b = e_v[pl.ds(h * _BPW + _HALF + k * _LANES, _LANES)]
            e = jnp.exp((b - a) * _C)
            e_v[pl.ds(h * _BPW + k * _LANES, _LANES)] = 1.0 / (1.0 + e)
        stores.append(pltpu.async_copy(
            e_v.at[pl.ds(h * _BPW, _HALF)],
            out_hbm.at[pl.ds(base + h * _HALF, _HALF)], osem))
    for c in stores:
        c.wait()


@jax.jit
def kernel(idx_a, idx_b, elos):
    mesh = plsc.VectorSubcoreMesh(core_axis_name="c", subcore_axis_name="s")
    run = functools.partial(
        pl.kernel,
        mesh=mesh,
        out_type=jax.ShapeDtypeStruct((_BATCH,), jnp.float32),
        scratch_types=[
            pltpu.VMEM((2 * _BPW,), jnp.int32),
            pltpu.VMEM((2 * _BPW,), jnp.float32),
            pltpu.SemaphoreType.DMA,
            pltpu.SemaphoreType.DMA,
            pltpu.SemaphoreType.DMA,
        ],
    )(_bt_body)
    a = idx_a.astype(jnp.int32).reshape(_NW, 2, _HALF)
    b = idx_b.astype(jnp.int32).reshape(_NW, 2, _HALF)
    idxc = jnp.stack([a, b], axis=2).reshape(-1)
    return run(idxc, elos)
